# baseline (speedup 1.0000x reference)
import functools

import jax
import jax.numpy as jnp
from jax import lax
from jax.experimental import pallas as pl
from jax.experimental.pallas import tpu as pltpu
from jax.experimental.pallas import tpu_sc as plsc

_D = 64
_NW = 32
_CH = 128
_K = 8
_ROWS = _CH * _K


def _build(n_tokens):
    per_w = n_tokens // _NW
    n_groups = per_w // _ROWS
    mesh = plsc.VectorSubcoreMesh(core_axis_name="c", subcore_axis_name="s")

    @functools.partial(
        pl.kernel,
        mesh=mesh,
        out_type=jax.ShapeDtypeStruct((n_tokens, _D), jnp.float32),
        scratch_types=[
            pltpu.VMEM((_ROWS,), jnp.int32),
            pltpu.VMEM((_ROWS, _D), jnp.float32),
            pltpu.SemaphoreType.DMA,
        ],
        compiler_params=pltpu.CompilerParams(use_tc_tiling_on_sc=False),
    )
    def emb(idx_hbm, tab_hbm, out_hbm, idx_v, rows_v, sem_g):
        wid = lax.axis_index("s") * 2 + lax.axis_index("c")
        base = wid * per_w

        def group(g, carry):
            gbase = base + g * _ROWS
            pltpu.sync_copy(idx_hbm.at[pl.ds(gbase, _ROWS)], idx_v)
            handles = [
                pltpu.async_copy(
                    tab_hbm.at[idx_v.at[pl.ds(j * _CH, _CH)]],
                    rows_v.at[pl.ds(j * _CH, _CH)],
                    sem_g,
                )
                for j in range(_K)
            ]
            for h in handles:
                h.wait()
            pltpu.sync_copy(rows_v, out_hbm.at[pl.ds(gbase, _ROWS)])
            return carry

        lax.fori_loop(0, n_groups, group, 0)

    return emb


def kernel(token_ids, weight):
    b, l = token_ids.shape
    n = b * l
    idx = token_ids.reshape(n).astype(jnp.int32)
    out = _build(n)(idx, weight)
    return out.reshape(b, l, _D)


# speedup vs baseline: 1.0168x; 1.0168x over previous
import functools

import jax
import jax.numpy as jnp
from jax import lax
from jax.experimental import pallas as pl
from jax.experimental.pallas import tpu as pltpu
from jax.experimental.pallas import tpu_sc as plsc

_D = 64
_NW = 32
_CH = 128
_K = 4
_ROWS = _CH * _K


def _build(n_tokens):
    per_w = n_tokens // _NW
    n_groups = per_w // _ROWS
    n_pairs = n_groups // 2
    mesh = plsc.VectorSubcoreMesh(core_axis_name="c", subcore_axis_name="s")

    @functools.partial(
        pl.kernel,
        mesh=mesh,
        out_type=jax.ShapeDtypeStruct((n_tokens, _D), jnp.float32),
        scratch_types=[
            pltpu.VMEM((per_w,), jnp.int32),
            pltpu.VMEM((2, _ROWS, _D), jnp.float32),
            pltpu.SemaphoreType.DMA,
            pltpu.SemaphoreType.DMA,
        ],
        compiler_params=pltpu.CompilerParams(use_tc_tiling_on_sc=False),
    )
    def emb(idx_hbm, tab_hbm, out_hbm, idx_v, rows_v, sem_g, sem_o):
        wid = lax.axis_index("s") * 2 + lax.axis_index("c")
        base = wid * per_w
        pltpu.sync_copy(idx_hbm.at[pl.ds(base, per_w)], idx_v)

        def fire(g, b):
            for j in range(_K):
                pltpu.async_copy(
                    tab_hbm.at[idx_v.at[pl.ds(g * _ROWS + j * _CH, _CH)]],
                    rows_v.at[b, pl.ds(j * _CH, _CH)],
                    sem_g,
                )

        def drain(b, sem):
            pltpu.make_async_copy(
                tab_hbm.at[pl.ds(0, _ROWS)], rows_v.at[b], sem
            ).wait()

        def start_out(g, b):
            pltpu.async_copy(
                rows_v.at[b], out_hbm.at[pl.ds(base + g * _ROWS, _ROWS)], sem_o
            )

        fire(0, 0)

        def pair(p, carry):
            for b in range(2):
                g = 2 * p + b
                nb = 1 - b

                @pl.when(g + 1 < n_groups)
                def _():
                    @pl.when(g >= 1)
                    def _():
                        drain(nb, sem_o)
                    fire(g + 1, nb)

                drain(b, sem_g)
                start_out(g, b)
            return carry

        lax.fori_loop(0, n_pairs, pair, 0)
        drain(0, sem_o)
        drain(1, sem_o)

    return emb


def kernel(token_ids, weight):
    b, l = token_ids.shape
    n = b * l
    idx = token_ids.reshape(n).astype(jnp.int32)
    out = _build(n)(idx, weight)
    return out.reshape(b, l, _D)


# speedup vs baseline: 1.0187x; 1.0020x over previous
import functools

import jax
import jax.numpy as jnp
from jax import lax
from jax.experimental import pallas as pl
from jax.experimental.pallas import tpu as pltpu
from jax.experimental.pallas import tpu_sc as plsc

_D = 64
_NW = 32
_T = 2
_SPLITS = ((0, 128), (128, 72))


def _build(b, l):
    rows_w = b // _NW
    n_groups = rows_w // _T
    n_pairs = n_groups // 2
    mesh = plsc.VectorSubcoreMesh(core_axis_name="c", subcore_axis_name="s")

    @functools.partial(
        pl.kernel,
        mesh=mesh,
        out_type=jax.ShapeDtypeStruct((b, l, _D), jnp.float32),
        scratch_types=[
            pltpu.VMEM((rows_w, l), jnp.int32),
            pltpu.VMEM((2, _T, l, _D), jnp.float32),
            pltpu.SemaphoreType.DMA,
            pltpu.SemaphoreType.DMA,
        ],
        compiler_params=pltpu.CompilerParams(use_tc_tiling_on_sc=False),
    )
    def emb(idx_hbm, tab_hbm, out_hbm, idx_v, rows_v, sem_g, sem_o):
        wid = lax.axis_index("s") * 2 + lax.axis_index("c")
        base = wid * rows_w
        pltpu.sync_copy(idx_hbm.at[pl.ds(base, rows_w)], idx_v)

        def fire(g, bf):
            for t in range(_T):
                for off, n in _SPLITS:
                    pltpu.async_copy(
                        tab_hbm.at[idx_v.at[g * _T + t, pl.ds(off, n)]],
                        rows_v.at[bf, t, pl.ds(off, n)],
                        sem_g,
                    )

        def drain(bf, sem):
            pltpu.make_async_copy(
                out_hbm.at[pl.ds(0, _T)], rows_v.at[bf], sem
            ).wait()

        def start_out(g, bf):
            pltpu.async_copy(
                rows_v.at[bf],
                out_hbm.at[pl.ds(base + g * _T, _T)],
                sem_o,
            )

        fire(0, 0)

        def pair(p, carry):
            for bf in range(2):
                g = 2 * p + bf
                nb = 1 - bf

                @pl.when(g + 1 < n_groups)
                def _():
                    @pl.when(g >= 1)
                    def _():
                        drain(nb, sem_o)
                    fire(g + 1, nb)

                drain(bf, sem_g)
                start_out(g, bf)
            return carry

        lax.fori_loop(0, n_pairs, pair, 0)
        drain(0, sem_o)
        drain(1, sem_o)

    return emb


def kernel(token_ids, weight):
    b, l = token_ids.shape
    return _build(b, l)(token_ids.astype(jnp.int32), weight)


# speedup vs baseline: 1.2457x; 1.2228x over previous
import functools

import jax
import jax.numpy as jnp
from jax import lax
from jax.experimental import pallas as pl
from jax.experimental.pallas import tpu as pltpu
from jax.experimental.pallas import tpu_sc as plsc

_DP = 128
_NW = 32
_T = 2
_SPLITS = ((0, 128), (128, 72))


def _build(b, l):
    rows_w = b // _NW
    n_groups = rows_w // _T
    n_pairs = n_groups // 2
    mesh = plsc.VectorSubcoreMesh(core_axis_name="c", subcore_axis_name="s")

    @functools.partial(
        pl.kernel,
        mesh=mesh,
        out_type=jax.ShapeDtypeStruct((b, l, _DP), jnp.float32),
        scratch_types=[
            pltpu.VMEM((2, _T, l), jnp.int32),
            pltpu.VMEM((2, _T, l, _DP), jnp.float32),
            pltpu.SemaphoreType.DMA,
            pltpu.SemaphoreType.DMA,
            pltpu.SemaphoreType.DMA,
        ],
    )
    def emb(idx_hbm, tab_hbm, out_hbm, idx_v, rows_v, sem_i, sem_g, sem_o):
        wid = lax.axis_index("s") * 2 + lax.axis_index("c")
        base = wid * rows_w

        def load_idx(g, bf):
            pltpu.async_copy(
                idx_hbm.at[pl.ds(base + g * _T, _T)], idx_v.at[bf], sem_i
            )

        def fire(g, bf):
            for t in range(_T):
                for off, n in _SPLITS:
                    pltpu.async_copy(
                        tab_hbm.at[idx_v.at[bf, t, pl.ds(off, n)]],
                        rows_v.at[bf, t, pl.ds(off, n)],
                        sem_g,
                    )

        def drain_rows(bf, sem):
            pltpu.make_async_copy(
                out_hbm.at[pl.ds(0, _T)], rows_v.at[bf], sem
            ).wait()

        def drain_idx(bf):
            pltpu.make_async_copy(
                idx_hbm.at[pl.ds(0, _T)], idx_v.at[bf], sem_i
            ).wait()

        def start_out(g, bf):
            pltpu.async_copy(
                rows_v.at[bf],
                out_hbm.at[pl.ds(base + g * _T, _T)],
                sem_o,
            )

        pltpu.sync_copy(idx_hbm.at[pl.ds(base, _T)], idx_v.at[0])
        fire(0, 0)
        load_idx(1, 1)

        def pair(p, carry):
            for bf in range(2):
                g = 2 * p + bf
                nb = 1 - bf

                @pl.when(g + 1 < n_groups)
                def _():
                    drain_idx(nb)

                    @pl.when(g >= 1)
                    def _():
                        drain_rows(nb, sem_o)

                    fire(g + 1, nb)

                drain_rows(bf, sem_g)
                start_out(g, bf)

                @pl.when(g + 2 < n_groups)
                def _():
                    load_idx(g + 2, bf)
            return carry

        lax.fori_loop(0, n_pairs, pair, 0)
        drain_rows(0, sem_o)
        drain_rows(1, sem_o)

    return emb


def kernel(token_ids, weight):
    b, l = token_ids.shape
    d = weight.shape[1]
    wpad = jnp.pad(weight, ((0, 0), (0, _DP - d)))
    raw = _build(b, l)(token_ids.astype(jnp.int32), wpad)
    return raw[:, :, :d]
